# baseline (speedup 1.0000x reference)
import functools

import jax
import jax.numpy as jnp
from jax import lax
from jax.experimental import pallas as pl
from jax.experimental.pallas import tpu as pltpu
from jax.experimental.pallas import tpu_sc as plsc

_SPLITS = 4
_LEN_SPLIT = 8
_SPLIT_EMBED = 16
_NUM_EMBEDDING = 1 << _LEN_SPLIT
_NUM_TABLES = 2 * _SPLITS
_N = 425984
_D = _NUM_TABLES * _SPLIT_EMBED

_NC, _NS, _L = 2, 16, 16
_NW = _NC * _NS
_PER_W = _N // _NW
_C = 256
_CHUNKS = _PER_W // _C
_CH = _C * _D
_TAB = _NUM_TABLES * _NUM_EMBEDDING * _SPLIT_EMBED


def _body(x_hbm, tab_hbm, out_hbm, tab_v, x_v, stg_v, sem):
  wid = lax.axis_index("s") * _NC + lax.axis_index("c")
  pltpu.sync_copy(tab_hbm, tab_v)
  pltpu.sync_copy(x_hbm.at[pl.ds(wid * _PER_W, _PER_W)], x_v)

  zeros = jnp.zeros((_L,), jnp.int32)
  ones = jnp.full((_L,), 1, jnp.int32)
  mask255 = jnp.full((_L,), (1 << _LEN_SPLIT) - 1, jnp.int32)
  four = jnp.full((_L,), 4, jnp.int32)
  iota128 = lax.iota(jnp.int32, _L) * _D
  out_base0 = wid * (_PER_W * _D)

  def chunk(g, carry):
    cur = lax.rem(g, 2)
    stg_off = cur * _CH

    @pl.when(g >= 2)
    def _drain():
      pltpu.make_async_copy(
          stg_v.at[pl.ds(stg_off, _CH)],
          out_hbm.at[pl.ds(out_base0, _CH)],
          sem,
      ).wait()

    def group(b, carry2):
      x = x_v[pl.ds(g * _C + b * _L, _L)]
      neg = x < zeros
      xa = jnp.abs(x)
      addr2 = iota128 + jnp.broadcast_to(stg_off + b * (_L * _D), (_L,))
      parts = []
      for i in range(_SPLITS):
        if i == 0:
          parts.append(xa & mask255)
        else:
          parts.append(
              lax.shift_right_arithmetic(
                  xa, jnp.full((_L,), 8 * i, jnp.int32)) & mask255)
      for t in range(_NUM_TABLES):
        p = parts[t % _SPLITS]
        part = jnp.where(neg, zeros, p) if t < _SPLITS else jnp.where(
            neg, p, zeros)
        addr = lax.shift_left(part, four) + (t * _NUM_EMBEDDING *
                                             _SPLIT_EMBED)
        for _ in range(_SPLIT_EMBED):
          val = plsc.load_gather(tab_v, [addr])
          plsc.store_scatter(stg_v, [addr2], val)
          addr = addr + ones
          addr2 = addr2 + ones
      return carry2

    lax.fori_loop(0, _C // _L, group, 0)
    pltpu.async_copy(
        stg_v.at[pl.ds(stg_off, _CH)],
        out_hbm.at[pl.ds(out_base0 + g * _CH, _CH)],
        sem,
    )
    return carry

  lax.fori_loop(0, _CHUNKS, chunk, 0)
  for _ in range(2):
    pltpu.make_async_copy(
        stg_v.at[pl.ds(0, _CH)],
        out_hbm.at[pl.ds(out_base0, _CH)],
        sem,
    ).wait()


_gather = functools.partial(
    pl.kernel,
    out_type=jax.ShapeDtypeStruct((_N * _D,), jnp.float32),
    mesh=plsc.VectorSubcoreMesh(core_axis_name="c", subcore_axis_name="s"),
    compiler_params=pltpu.CompilerParams(
        needs_layout_passes=False, use_tc_tiling_on_sc=False),
    scratch_types=[
        pltpu.VMEM((_TAB,), jnp.float32),
        pltpu.VMEM((_PER_W,), jnp.int32),
        pltpu.VMEM((2 * _CH,), jnp.float32),
        pltpu.SemaphoreType.DMA,
    ],
)(_body)


@jax.jit
def kernel(X, tables):
  out = _gather(X, tables.reshape(-1))
  return out.reshape(_N, _D)


# speedup vs baseline: 1.4055x; 1.4055x over previous
import functools

import jax
import jax.numpy as jnp
from jax import lax
from jax.experimental import pallas as pl
from jax.experimental.pallas import tpu as pltpu
from jax.experimental.pallas import tpu_sc as plsc

_SPLITS = 4
_LEN_SPLIT = 8
_SPLIT_EMBED = 16
_NUM_EMBEDDING = 1 << _LEN_SPLIT
_NUM_TABLES = 2 * _SPLITS
_N = 425984
_D = _NUM_TABLES * _SPLIT_EMBED

_NC, _NS, _L = 2, 16, 16
_NW = _NC * _NS
_PER_W = _N // _NW
_C = 256
_CHUNKS = _PER_W // _C
_CH = _C * _D
_TAB = _NUM_TABLES * _NUM_EMBEDDING * _SPLIT_EMBED


def _body(x_hbm, tab_hbm, out_hbm, tab_v, x_v, stg_v, sem):
  wid = lax.axis_index("s") * _NC + lax.axis_index("c")
  pltpu.sync_copy(tab_hbm, tab_v)
  pltpu.sync_copy(x_hbm.at[pl.ds(wid * _PER_W, _PER_W)], x_v)

  zeros = jnp.zeros((_L,), jnp.int32)
  mask255 = jnp.full((_L,), (1 << _LEN_SPLIT) - 1, jnp.int32)
  four = jnp.full((_L,), 4, jnp.int32)
  sixteen = jnp.full((_L,), _SPLIT_EMBED, jnp.int32)
  iota128 = lax.iota(jnp.int32, _L) * _D
  offs = [jnp.full((_L,), j, jnp.int32) for j in range(_SPLIT_EMBED)]
  all_true = jnp.full((_L,), True, jnp.bool_)
  out_base0 = wid * (_PER_W * _D)

  def chunk(g, carry):
    cur = lax.rem(g, 2)
    stg_off = cur * _CH

    @pl.when(g >= 2)
    def _drain():
      pltpu.make_async_copy(
          stg_v.at[pl.ds(stg_off, _CH)],
          out_hbm.at[pl.ds(out_base0, _CH)],
          sem,
      ).wait()

    def group(b, carry2):
      x = x_v[pl.ds(g * _C + b * _L, _L)]
      neg = x < zeros
      xa = jnp.abs(x)
      addr2 = iota128 + jnp.broadcast_to(stg_off + b * (_L * _D), (_L,))
      parts = []
      for i in range(_SPLITS):
        if i == 0:
          parts.append(xa & mask255)
        else:
          parts.append(
              lax.shift_right_arithmetic(
                  xa, jnp.full((_L,), 8 * i, jnp.int32)) & mask255)
      for t in range(_NUM_TABLES):
        p = parts[t % _SPLITS]
        part = jnp.where(neg, zeros, p) if t < _SPLITS else jnp.where(
            neg, p, zeros)
        base = lax.shift_left(part, four) + (t * _NUM_EMBEDDING *
                                             _SPLIT_EMBED)
        vals = [
            plsc.load_gather(tab_v, [base + offs[j]], mask=all_true)
            for j in range(_SPLIT_EMBED)
        ]
        for j in range(_SPLIT_EMBED):
          plsc.store_scatter(stg_v, [addr2 + offs[j]], vals[j],
                             mask=all_true)
        addr2 = addr2 + sixteen
      return carry2

    lax.fori_loop(0, _C // _L, group, 0)
    pltpu.async_copy(
        stg_v.at[pl.ds(stg_off, _CH)],
        out_hbm.at[pl.ds(out_base0 + g * _CH, _CH)],
        sem,
    )
    return carry

  lax.fori_loop(0, _CHUNKS, chunk, 0)
  for _ in range(2):
    pltpu.make_async_copy(
        stg_v.at[pl.ds(0, _CH)],
        out_hbm.at[pl.ds(out_base0, _CH)],
        sem,
    ).wait()


_gather = functools.partial(
    pl.kernel,
    out_type=jax.ShapeDtypeStruct((_N * _D,), jnp.float32),
    mesh=plsc.VectorSubcoreMesh(core_axis_name="c", subcore_axis_name="s"),
    compiler_params=pltpu.CompilerParams(
        needs_layout_passes=False, use_tc_tiling_on_sc=False),
    scratch_types=[
        pltpu.VMEM((_TAB,), jnp.float32),
        pltpu.VMEM((_PER_W,), jnp.int32),
        pltpu.VMEM((2 * _CH,), jnp.float32),
        pltpu.SemaphoreType.DMA,
    ],
)(_body)


@jax.jit
def kernel(X, tables):
  out = _gather(X, tables.reshape(-1))
  return out.reshape(_N, _D)


# speedup vs baseline: 3.1699x; 2.2554x over previous
import functools

import jax
import jax.numpy as jnp
from jax import lax
from jax.experimental import pallas as pl
from jax.experimental.pallas import tpu as pltpu
from jax.experimental.pallas import tpu_sc as plsc

_SPLITS = 4
_LEN_SPLIT = 8
_SPLIT_EMBED = 16
_NUM_EMBEDDING = 1 << _LEN_SPLIT
_NUM_TABLES = 2 * _SPLITS
_N = 425984
_D = _NUM_TABLES * _SPLIT_EMBED

_NC, _NS, _L = 2, 16, 16
_NW = _NC * _NS
_PER_W = _N // _NW
_C = 256
_CHUNKS = _PER_W // _C
_CH = _C * _D
_TAB = _NUM_TABLES * _NUM_EMBEDDING * _SPLIT_EMBED


def _body(x_hbm, tab_hbm, out_hbm, tab_v, x_v, stg_v, sem):
  wid = lax.axis_index("s") * _NC + lax.axis_index("c")
  pltpu.sync_copy(tab_hbm, tab_v)
  pltpu.sync_copy(x_hbm.at[pl.ds(wid * _PER_W, _PER_W)], x_v)

  zeros = jnp.zeros((_L,), jnp.int32)
  mask255 = jnp.full((_L,), (1 << _LEN_SPLIT) - 1, jnp.int32)
  four = jnp.full((_L,), 4, jnp.int32)
  iota16 = lax.iota(jnp.int32, _L)
  offs = [jnp.full((_L,), j, jnp.int32) for j in range(_L)]
  all_true = jnp.full((_L,), True, jnp.bool_)
  out_base0 = wid * (_PER_W * _D)

  def chunk(g, carry):
    cur = lax.rem(g, 2)
    stg_off = cur * _CH

    @pl.when(g >= 2)
    def _drain():
      pltpu.make_async_copy(
          stg_v.at[pl.ds(stg_off, _CH)],
          out_hbm.at[pl.ds(out_base0, _CH)],
          sem,
      ).wait()

    def group(b, carry2):
      x = x_v[pl.ds(g * _C + b * _L, _L)]
      neg = x < zeros
      xa = jnp.abs(x)
      off0 = stg_off + b * (_L * _D)
      parts = []
      for i in range(_SPLITS):
        if i == 0:
          parts.append(xa & mask255)
        else:
          parts.append(
              lax.shift_right_arithmetic(
                  xa, jnp.full((_L,), 8 * i, jnp.int32)) & mask255)
      bases = []
      for t in range(_NUM_TABLES):
        p = parts[t % _SPLITS]
        part = jnp.where(neg, zeros, p) if t < _SPLITS else jnp.where(
            neg, p, zeros)
        bases.append(
            lax.shift_left(part, four) + (t * _NUM_EMBEDDING * _SPLIT_EMBED))
      for t in range(_NUM_TABLES):
        for e in range(_L):
          sp = bases[t].at[offs[e]].get(mode="promise_in_bounds")
          val = plsc.load_gather(tab_v, [sp + iota16], mask=all_true)
          stg_v[pl.ds(off0 + e * _D + t * _SPLIT_EMBED, _SPLIT_EMBED)] = val
      return carry2

    lax.fori_loop(0, _C // _L, group, 0)
    pltpu.async_copy(
        stg_v.at[pl.ds(stg_off, _CH)],
        out_hbm.at[pl.ds(out_base0 + g * _CH, _CH)],
        sem,
    )
    return carry

  lax.fori_loop(0, _CHUNKS, chunk, 0)
  for _ in range(2):
    pltpu.make_async_copy(
        stg_v.at[pl.ds(0, _CH)],
        out_hbm.at[pl.ds(out_base0, _CH)],
        sem,
    ).wait()


_gather = functools.partial(
    pl.kernel,
    out_type=jax.ShapeDtypeStruct((_N * _D,), jnp.float32),
    mesh=plsc.VectorSubcoreMesh(core_axis_name="c", subcore_axis_name="s"),
    compiler_params=pltpu.CompilerParams(
        needs_layout_passes=False, use_tc_tiling_on_sc=False),
    scratch_types=[
        pltpu.VMEM((_TAB,), jnp.float32),
        pltpu.VMEM((_PER_W,), jnp.int32),
        pltpu.VMEM((2 * _CH,), jnp.float32),
        pltpu.SemaphoreType.DMA,
    ],
)(_body)


@jax.jit
def kernel(X, tables):
  out = _gather(X, tables.reshape(-1))
  return out.reshape(_N, _D)
